# baseline (speedup 1.0000x reference)
import functools

import jax
import jax.numpy as jnp
from jax import lax
from jax.experimental import pallas as pl
from jax.experimental.pallas import tpu as pltpu
from jax.experimental.pallas import tpu_sc as plsc

_B = 16384
_DIM = 128
_NC = 2
_NS = 16
_NW = _NC * _NS
_CHUNKS = (12288, 4096)

_sc_mesh = plsc.VectorSubcoreMesh(core_axis_name="c", subcore_axis_name="s")


def _make_sc_gather(chunk_off, chunk_rows):
    bpw = chunk_rows // _NW

    @functools.partial(
        pl.kernel,
        mesh=_sc_mesh,
        out_type=jax.ShapeDtypeStruct((chunk_rows, _DIM), jnp.float32),
        scratch_types=[
            pltpu.VMEM((bpw,), jnp.int32),
            pltpu.VMEM((bpw, _DIM), jnp.float32),
            pltpu.SemaphoreType.DMA,
        ],
    )
    def _sc_gather(table_hbm, idx_hbm, out_hbm, idx_v, rows_v, sem):
        wid = lax.axis_index("s") * _NC + lax.axis_index("c")
        base = wid * bpw
        pltpu.sync_copy(idx_hbm.at[pl.ds(chunk_off + base, bpw)], idx_v)
        pltpu.async_copy(table_hbm.at[idx_v], rows_v, sem).wait()
        pltpu.sync_copy(rows_v, out_hbm.at[pl.ds(base, bpw)])

    return _sc_gather


_off = [sum(_CHUNKS[:k]) for k in range(len(_CHUNKS))]
_sc_gathers = [_make_sc_gather(_off[k], _CHUNKS[k]) for k in range(len(_CHUNKS))]


def _tc_body(x_ref, w_ref, b_ref, o_ref):
    acc = lax.dot_general(
        x_ref[...], w_ref[...],
        dimension_numbers=(((1,), (1,)), ((), ())),
        preferred_element_type=jnp.float32,
    )
    o_ref[...] = jnp.tanh(acc + b_ref[...])


def _tc_body_alias(x_ref, w_ref, b_ref, prev_ref, o_ref):
    del prev_ref
    _tc_body(x_ref, w_ref, b_ref, o_ref)


def _tc_chunk(x, w, b2d, prev, row_off, blk):
    rows = x.shape[0]
    nblk = rows // blk
    off = row_off // blk
    in_specs = [
        pl.BlockSpec((blk, _DIM), lambda i: (i, 0)),
        pl.BlockSpec((_DIM, _DIM), lambda i: (0, 0)),
        pl.BlockSpec((1, _DIM), lambda i: (0, 0)),
    ]
    args = (x, w, b2d)
    if prev is None:
        body, aliases = _tc_body, {}
    else:
        body, aliases = _tc_body_alias, {3: 0}
        in_specs.append(pl.BlockSpec(memory_space=pl.ANY))
        args = args + (prev,)
    return pl.pallas_call(
        body,
        grid=(nblk,),
        in_specs=in_specs,
        out_specs=pl.BlockSpec((blk, _DIM), lambda i, off=off: (i + off, 0)),
        out_shape=jax.ShapeDtypeStruct((_B, _DIM), jnp.float32),
        input_output_aliases=aliases,
    )(*args)


def kernel(nodes, embeddings, W1_w, W1_b):
    idx = nodes.astype(jnp.int32)
    b2d = W1_b.reshape(1, _DIM)
    gathered = [g(embeddings, idx) for g in _sc_gathers]
    out = None
    for k in range(len(_CHUNKS)):
        out = _tc_chunk(gathered[k], W1_w, b2d, out, _off[k], _CHUNKS[k] // 2)
    return out
